# baseline (speedup 1.0000x reference)
import jax
import jax.numpy as jnp
from jax import lax
from jax.experimental import pallas as pl
from jax.experimental.pallas import tpu as pltpu
from jax.experimental.pallas import tpu_sc as plsc

DIM = 64
N_IDX = 16384
NC, NS = 2, 16
NW = NC * NS
B_PER_W = N_IDX // NW


def _gather_body(idx_hbm, table_hbm, out_hbm, idx_v, rows_v, sem):
    wid = lax.axis_index("s") * NC + lax.axis_index("c")
    base = wid * B_PER_W
    pltpu.sync_copy(idx_hbm.at[pl.ds(base, B_PER_W)],
                    idx_v.at[pl.ds(0, B_PER_W)])

    def body(n, _):
        r = idx_v[pl.ds(n, 16)][0]
        pltpu.make_async_copy(table_hbm.at[r], rows_v.at[n], sem).start()
        return 0

    lax.fori_loop(0, B_PER_W, body, 0)
    pltpu.make_async_copy(table_hbm.at[pl.ds(0, B_PER_W)], rows_v, sem).wait()
    pltpu.sync_copy(rows_v, out_hbm.at[pl.ds(base, B_PER_W)])


@jax.jit
def _gather(indics, table):
    mesh = plsc.VectorSubcoreMesh(core_axis_name="c", subcore_axis_name="s")
    return pl.kernel(
        _gather_body,
        out_type=jax.ShapeDtypeStruct((N_IDX, DIM), jnp.float32),
        mesh=mesh,
        scratch_types=[
            pltpu.VMEM((B_PER_W + 16,), jnp.int32),
            pltpu.VMEM((B_PER_W, DIM), jnp.float32),
            pltpu.SemaphoreType.DMA,
        ],
    )(indics, table)


def kernel(indics, offset, weight):
    del offset
    return _gather(indics, weight)


# speedup vs baseline: 1.4789x; 1.4789x over previous
import jax
import jax.numpy as jnp
from jax import lax
from jax.experimental import pallas as pl
from jax.experimental.pallas import tpu as pltpu
from jax.experimental.pallas import tpu_sc as plsc

DIM = 64
N_IDX = 16384
TILE_R = 8
NC, NS = 2, 16
NW = NC * NS
B_PER_W = N_IDX // NW


def _gather_body(idx_hbm, table_hbm, out_hbm, idx_v, rows_v, sem):
    wid = lax.axis_index("s") * NC + lax.axis_index("c")
    base = wid * B_PER_W
    pltpu.sync_copy(idx_hbm.at[pl.ds(base, B_PER_W)],
                    idx_v.at[pl.ds(0, B_PER_W)])

    def body(n, _):
        r = idx_v[pl.ds(n, 16)][0]
        pltpu.make_async_copy(
            table_hbm.at[r >> 3, r & (TILE_R - 1)],
            rows_v.at[n // TILE_R, n % TILE_R],
            sem,
        ).start()
        return 0

    lax.fori_loop(0, B_PER_W, body, 0)
    pltpu.make_async_copy(table_hbm.at[pl.ds(0, B_PER_W // TILE_R)],
                          rows_v, sem).wait()
    pltpu.sync_copy(rows_v,
                    out_hbm.at[pl.ds(base // TILE_R, B_PER_W // TILE_R)])


@jax.jit
def _gather(indics, table3):
    mesh = plsc.VectorSubcoreMesh(core_axis_name="c", subcore_axis_name="s")
    return pl.kernel(
        _gather_body,
        out_type=jax.ShapeDtypeStruct((N_IDX // TILE_R, TILE_R, DIM),
                                      jnp.float32),
        mesh=mesh,
        scratch_types=[
            pltpu.VMEM((B_PER_W + 16,), jnp.int32),
            pltpu.VMEM((B_PER_W // TILE_R, TILE_R, DIM), jnp.float32),
            pltpu.SemaphoreType.DMA,
        ],
    )(indics, table3)


def kernel(indics, offset, weight):
    del offset
    table3 = weight.reshape(weight.shape[0] // TILE_R, TILE_R, DIM)
    out3 = _gather(indics, table3)
    return out3.reshape(N_IDX, DIM)


# speedup vs baseline: 1.5714x; 1.0626x over previous
import jax
import jax.numpy as jnp
from jax import lax
from jax.experimental import pallas as pl
from jax.experimental.pallas import tpu as pltpu
from jax.experimental.pallas import tpu_sc as plsc

DIM = 64
N_IDX = 16384
NUM = 1000000
NC, NS = 2, 16
NW = NC * NS
NBLK = NUM // 128
TAIL = NUM - NBLK * 128
BASE_BLKS = NBLK // NW
EXTRA = NBLK - BASE_BLKS * NW
CAP = N_IDX


def _body(idx_hbm, table_hbm, out_hbm,
          idx_v, cidx, cpos, gidx, gpos, hist, off, off1,
          ring, tailbuf, stage,
          sem_a, sem_b, sem_o):
    w = lax.axis_index("s") * NC + lax.axis_index("c")
    myfirst = w * BASE_BLKS + jnp.minimum(w, EXTRA)
    nfull = BASE_BLKS + jnp.where(w < EXTRA, 1, 0)
    lo = myfirst * 128
    hi = jnp.where(w == NW - 1, NUM, (myfirst + nfull) * 128)

    pltpu.sync_copy(idx_hbm, idx_v.at[pl.ds(0, N_IDX)])

    iota = lax.iota(jnp.int32, 16)
    lane0 = iota == 0

    def filt(k, cur):
        v = idx_v[pl.ds(k * 16, 16)]
        m = (v >= lo) & (v < hi)
        mi = jnp.where(m, 1, 0)
        pc = plsc.cumsum(mi)
        tgt = jnp.where(m, cur + pc - mi, CAP + 8)
        plsc.store_scatter(cidx, [tgt], v)
        plsc.store_scatter(cpos, [tgt], k * 16 + iota)
        return cur + pc[15]

    nmine = lax.fori_loop(0, N_IDX // 16, filt, 0)

    zeros = jnp.zeros((16,), jnp.int32)

    def zero(k, _):
        hist[pl.ds(k * 16, 16)] = zeros
        off[pl.ds(k * 16, 16)] = zeros
        return 0
    lax.fori_loop(0, 256 // 16, zero, 0)

    ones = jnp.ones((16,), jnp.int32)

    def count(k, _):
        c = (cidx[pl.ds(k, 16)][0] >> 7) - myfirst
        plsc.addupdate_scatter(hist, [jnp.where(lane0, c, 255)], ones)
        return 0
    lax.fori_loop(0, nmine, count, 0)

    def prefix(k, carry):
        s = plsc.cumsum(hist[pl.ds(k * 16, 16)])
        off[pl.ds(k * 16 + 1, 16)] = s + carry
        return carry + s[15]
    lax.fori_loop(0, 256 // 16 - 1, prefix, 0)

    def cpy(k, _):
        off1[pl.ds(k * 16, 16)] = off[pl.ds(k * 16, 16)]
        return 0
    lax.fori_loop(0, 256 // 16, cpy, 0)

    def scat(k, _):
        v = cidx[pl.ds(k, 16)][0]
        p = cpos[pl.ds(k, 16)][0]
        c = (v >> 7) - myfirst
        o = off1[pl.ds(c, 16)][0]
        tgt = jnp.where(lane0, o, CAP + 8)
        plsc.store_scatter(gidx, [tgt], jnp.broadcast_to(v, (16,)))
        plsc.store_scatter(gpos, [tgt], jnp.broadcast_to(p, (16,)))
        plsc.store_scatter(off1, [jnp.where(lane0, c, 260)],
                           jnp.broadcast_to(o + 1, (16,)))
        return 0
    lax.fori_loop(0, nmine, scat, 0)

    def prime(s, _):
        pltpu.make_async_copy(table_hbm.at[0, pl.ds(0, DIM)],
                              stage.at[s], sem_o).start()
        return 0
    lax.fori_loop(0, 8, prime, 0)

    def start_chunk(c, buf, sem):
        src = table_hbm.at[:, pl.ds(pl.multiple_of((myfirst + c) * 128, 128),
                                    128)]
        pltpu.make_async_copy(src, buf, sem).start()

    def process(par_const, c):
        def ent(k, _):
            v = gidx[pl.ds(k, 16)][0]
            p = gpos[pl.ds(k, 16)][0]
            col = v & 127
            slot = k & 7
            pltpu.make_async_copy(table_hbm.at[0, pl.ds(0, DIM)],
                                  stage.at[slot], sem_o).wait()
            for g in range(DIM // 16):
                vec = plsc.load_gather(
                    ring, [jnp.broadcast_to(par_const, (16,)),
                           iota + g * 16,
                           jnp.broadcast_to(col, (16,))])
                stage[slot, pl.ds(g * 16, 16)] = vec
            pltpu.make_async_copy(stage.at[slot],
                                  out_hbm.at[pl.ds(p * DIM, DIM)],
                                  sem_o).start()
            return 0
        lax.fori_loop(off[pl.ds(c, 16)][0], off[pl.ds(c + 1, 16)][0], ent, 0)

    start_chunk(0, ring.at[0], sem_a)
    start_chunk(1, ring.at[1], sem_b)

    def pair(pp, _):
        c0 = pp * 2
        pltpu.make_async_copy(table_hbm.at[:, pl.ds(0, 128)],
                              ring.at[0], sem_a).wait()
        process(0, c0)

        @pl.when(c0 + 2 < nfull)
        def _():
            start_chunk(c0 + 2, ring.at[0], sem_a)

        pltpu.make_async_copy(table_hbm.at[:, pl.ds(0, 128)],
                              ring.at[1], sem_b).wait()
        process(1, c0 + 1)

        @pl.when(c0 + 3 < nfull)
        def _():
            start_chunk(c0 + 3, ring.at[1], sem_b)
        return 0

    lax.fori_loop(0, BASE_BLKS // 2, pair, 0)

    @pl.when(nfull > BASE_BLKS)
    def _():
        pltpu.make_async_copy(table_hbm.at[:, pl.ds(0, 128)],
                              ring.at[0], sem_a).wait()
        process(0, BASE_BLKS)

    @pl.when(w == NW - 1)
    def _():
        pltpu.sync_copy(table_hbm.at[:, pl.ds(NBLK * 128, TAIL)], tailbuf)

        def ent(k, _):
            v = gidx[pl.ds(k, 16)][0]
            p = gpos[pl.ds(k, 16)][0]
            col = v & 127
            slot = k & 7
            pltpu.make_async_copy(table_hbm.at[0, pl.ds(0, DIM)],
                                  stage.at[slot], sem_o).wait()
            for g in range(DIM // 16):
                vec = plsc.load_gather(
                    tailbuf, [iota + g * 16, jnp.broadcast_to(col, (16,))])
                stage[slot, pl.ds(g * 16, 16)] = vec
            pltpu.make_async_copy(stage.at[slot],
                                  out_hbm.at[pl.ds(p * DIM, DIM)],
                                  sem_o).start()
            return 0
        lax.fori_loop(off[pl.ds(BASE_BLKS, 16)][0],
                      off[pl.ds(BASE_BLKS + 1, 16)][0], ent, 0)

    def drain(s, _):
        pltpu.make_async_copy(table_hbm.at[0, pl.ds(0, DIM)],
                              stage.at[s], sem_o).wait()
        return 0
    lax.fori_loop(0, 8, drain, 0)


@jax.jit
def _gather(indics, table_t):
    mesh = plsc.VectorSubcoreMesh(core_axis_name="c", subcore_axis_name="s")
    return pl.kernel(
        _body,
        out_type=jax.ShapeDtypeStruct((N_IDX * DIM,), jnp.float32),
        mesh=mesh,
        compiler_params=pltpu.CompilerParams(needs_layout_passes=False),
        scratch_types=[
            pltpu.VMEM((N_IDX + 16,), jnp.int32),
            pltpu.VMEM((CAP + 16,), jnp.int32),
            pltpu.VMEM((CAP + 16,), jnp.int32),
            pltpu.VMEM((CAP + 16,), jnp.int32),
            pltpu.VMEM((CAP + 16,), jnp.int32),
            pltpu.VMEM((256,), jnp.int32),
            pltpu.VMEM((272,), jnp.int32),
            pltpu.VMEM((272,), jnp.int32),
            pltpu.VMEM((2, DIM, 128), jnp.float32),
            pltpu.VMEM((DIM, TAIL), jnp.float32),
            pltpu.VMEM((8, DIM), jnp.float32),
            pltpu.SemaphoreType.DMA,
            pltpu.SemaphoreType.DMA,
            pltpu.SemaphoreType.DMA,
        ],
    )(indics, table_t)


def kernel(indics, offset, weight):
    del offset
    return _gather(indics, weight.T).reshape(N_IDX, DIM)


# speedup vs baseline: 1.8478x; 1.1759x over previous
import jax
import jax.numpy as jnp
from jax import lax
from jax.experimental import pallas as pl
from jax.experimental.pallas import tpu as pltpu
from jax.experimental.pallas import tpu_sc as plsc

DIM = 64
N_IDX = 16384
NUM = 1000000
NC, NS = 2, 16
NW = NC * NS
NBLK = NUM // 128
TAIL = NUM - NBLK * 128
BASE_BLKS = NBLK // NW
EXTRA = NBLK - BASE_BLKS * NW
CAP = N_IDX


def _body(idx_hbm, table_hbm, out_hbm,
          idx_v, cidx, cpos, gidx, gpos, hist, off, off1,
          ring, tailbuf, stage,
          sem_a, sem_b, sem_c, sem_d, sem_o):
    w = lax.axis_index("s") * NC + lax.axis_index("c")
    myfirst = w * BASE_BLKS + jnp.minimum(w, EXTRA)
    nfull = BASE_BLKS + jnp.where(w < EXTRA, 1, 0)
    lo = myfirst * 128
    hi = jnp.where(w == NW - 1, NUM, (myfirst + nfull) * 128)

    csems = (sem_a, sem_b, sem_c, sem_d)

    def start_chunk(c, j):
        src = table_hbm.at[:, pl.ds(pl.multiple_of((myfirst + c) * 128, 128),
                                    128)]
        pltpu.make_async_copy(src, ring.at[j], csems[j]).start()

    for j in range(4):
        start_chunk(j, j)

    def prime(s, _):
        pltpu.make_async_copy(table_hbm.at[0, pl.ds(0, DIM)],
                              stage.at[s], sem_o).start()
        return 0
    lax.fori_loop(0, 32, prime, 0)

    pltpu.sync_copy(idx_hbm, idx_v.at[pl.ds(0, N_IDX)])

    iota = lax.iota(jnp.int32, 16)
    lane0 = iota == 0

    def filt(k, cur):
        v = idx_v[pl.ds(k * 16, 16)]
        m = (v >= lo) & (v < hi)
        mi = jnp.where(m, 1, 0)
        pc = plsc.cumsum(mi)
        tgt = jnp.where(m, cur + pc - mi, CAP + 8)
        plsc.store_scatter(cidx, [tgt], v)
        plsc.store_scatter(cpos, [tgt], k * 16 + iota)
        return cur + pc[15]

    nmine = lax.fori_loop(0, N_IDX // 16, filt, 0)

    zeros = jnp.zeros((16,), jnp.int32)

    def zero(k, _):
        hist[pl.ds(k * 16, 16)] = zeros
        off[pl.ds(k * 16, 16)] = zeros
        return 0
    lax.fori_loop(0, 256 // 16, zero, 0)

    ones = jnp.ones((16,), jnp.int32)

    def count(k, _):
        c = (cidx[pl.ds(k, 16)][0] >> 7) - myfirst
        plsc.addupdate_scatter(hist, [jnp.where(lane0, c, 255)], ones)
        return 0
    lax.fori_loop(0, nmine, count, 0)

    def prefix(k, carry):
        s = plsc.cumsum(hist[pl.ds(k * 16, 16)])
        off[pl.ds(k * 16 + 1, 16)] = s + carry
        return carry + s[15]
    lax.fori_loop(0, 256 // 16 - 1, prefix, 0)

    def cpy(k, _):
        off1[pl.ds(k * 16, 16)] = off[pl.ds(k * 16, 16)]
        return 0
    lax.fori_loop(0, 256 // 16, cpy, 0)

    def scat(k, _):
        v = cidx[pl.ds(k, 16)][0]
        p = cpos[pl.ds(k, 16)][0]
        c = (v >> 7) - myfirst
        o = off1[pl.ds(c, 16)][0]
        tgt = jnp.where(lane0, o, CAP + 8)
        plsc.store_scatter(gidx, [tgt], jnp.broadcast_to(v, (16,)))
        plsc.store_scatter(gpos, [tgt], jnp.broadcast_to(p, (16,)))
        plsc.store_scatter(off1, [jnp.where(lane0, c, 260)],
                           jnp.broadcast_to(o + 1, (16,)))
        return 0
    lax.fori_loop(0, nmine, scat, 0)

    def process(par_const, c):
        def ent(k, _):
            v = gidx[pl.ds(k, 16)][0]
            p = gpos[pl.ds(k, 16)][0]
            col = v & 127
            slot = k & 31
            pltpu.make_async_copy(table_hbm.at[0, pl.ds(0, DIM)],
                                  stage.at[slot], sem_o).wait()
            for g in range(DIM // 16):
                vec = plsc.load_gather(
                    ring, [jnp.broadcast_to(par_const, (16,)),
                           iota + g * 16,
                           jnp.broadcast_to(col, (16,))])
                stage[slot, pl.ds(g * 16, 16)] = vec
            pltpu.make_async_copy(stage.at[slot],
                                  out_hbm.at[pl.ds(p * DIM, DIM)],
                                  sem_o).start()
            return 0
        lax.fori_loop(off[pl.ds(c, 16)][0], off[pl.ds(c + 1, 16)][0], ent, 0)

    def quad(qq, _):
        c0 = qq * 4
        for j in range(4):
            pltpu.make_async_copy(table_hbm.at[:, pl.ds(0, 128)],
                                  ring.at[j], csems[j]).wait()
            process(j, c0 + j)

            @pl.when(c0 + j + 4 < nfull)
            def _():
                start_chunk(c0 + j + 4, j)
        return 0

    lax.fori_loop(0, BASE_BLKS // 4, quad, 0)

    @pl.when(nfull > BASE_BLKS)
    def _():
        pltpu.make_async_copy(table_hbm.at[:, pl.ds(0, 128)],
                              ring.at[0], sem_a).wait()
        process(0, BASE_BLKS)

    @pl.when(w == NW - 1)
    def _():
        pltpu.sync_copy(table_hbm.at[:, pl.ds(NBLK * 128, TAIL)], tailbuf)

        def ent(k, _):
            v = gidx[pl.ds(k, 16)][0]
            p = gpos[pl.ds(k, 16)][0]
            col = v & 127
            slot = k & 31
            pltpu.make_async_copy(table_hbm.at[0, pl.ds(0, DIM)],
                                  stage.at[slot], sem_o).wait()
            for g in range(DIM // 16):
                vec = plsc.load_gather(
                    tailbuf, [iota + g * 16, jnp.broadcast_to(col, (16,))])
                stage[slot, pl.ds(g * 16, 16)] = vec
            pltpu.make_async_copy(stage.at[slot],
                                  out_hbm.at[pl.ds(p * DIM, DIM)],
                                  sem_o).start()
            return 0
        lax.fori_loop(off[pl.ds(BASE_BLKS, 16)][0],
                      off[pl.ds(BASE_BLKS + 1, 16)][0], ent, 0)

    def drain(s, _):
        pltpu.make_async_copy(table_hbm.at[0, pl.ds(0, DIM)],
                              stage.at[s], sem_o).wait()
        return 0
    lax.fori_loop(0, 32, drain, 0)


@jax.jit
def _gather(indics, table_t):
    mesh = plsc.VectorSubcoreMesh(core_axis_name="c", subcore_axis_name="s")
    return pl.kernel(
        _body,
        out_type=jax.ShapeDtypeStruct((N_IDX * DIM,), jnp.float32),
        mesh=mesh,
        compiler_params=pltpu.CompilerParams(needs_layout_passes=False),
        scratch_types=[
            pltpu.VMEM((N_IDX + 16,), jnp.int32),
            pltpu.VMEM((CAP + 16,), jnp.int32),
            pltpu.VMEM((CAP + 16,), jnp.int32),
            pltpu.VMEM((CAP + 16,), jnp.int32),
            pltpu.VMEM((CAP + 16,), jnp.int32),
            pltpu.VMEM((256,), jnp.int32),
            pltpu.VMEM((272,), jnp.int32),
            pltpu.VMEM((272,), jnp.int32),
            pltpu.VMEM((4, DIM, 128), jnp.float32),
            pltpu.VMEM((DIM, TAIL), jnp.float32),
            pltpu.VMEM((32, DIM), jnp.float32),
            pltpu.SemaphoreType.DMA,
            pltpu.SemaphoreType.DMA,
            pltpu.SemaphoreType.DMA,
            pltpu.SemaphoreType.DMA,
            pltpu.SemaphoreType.DMA,
        ],
    )(indics, table_t)


def kernel(indics, offset, weight):
    del offset
    return _gather(indics, weight.T).reshape(N_IDX, DIM)
